# baseline (speedup 1.0000x reference)
import functools

import jax
import jax.numpy as jnp
from jax import lax
from jax.experimental import pallas as pl
from jax.experimental.pallas import tpu as pltpu
from jax.experimental.pallas import tpu_sc as plsc

N_NODES = 10000
N_EDGES = 320000
D = 768

NSLICE = 6
DSL = D // NSLICE
EB = 100
NTILES = 16
ROWS_PER_TILE = N_EDGES // EB // NTILES
CHUNK = 8
NCHUNK = ROWS_PER_TILE // CHUNK
NPAIR = (NCHUNK - 1) // 2
ZROWS = 624
ZLAST = N_NODES - 15 * ZROWS
MM_BLK = 400
MM_GRID = N_NODES // MM_BLK
HROWS = 80

_sc_mesh = plsc.VectorSubcoreMesh(core_axis_name="c", subcore_axis_name="s", num_cores=1)


def _agg_body(mode_hbm, y0, y1, y2, y3, y4, y5, src_hbm, dst_hbm, zero_hbm,
              ones_hbm, o0, o1, o2, o3, o4, o5,
              mode_s, srcA, dstA, srcB, dstB, rows0, rows1, rows2,
              gsem0, gsem1, gsem2, ssem0, ssem1, ssem2, isemA, isemB, acc):
    core = lax.axis_index("c")
    sub = lax.axis_index("s")

    pltpu.sync_copy(mode_hbm, mode_s)
    deg_mode = mode_s[pl.ds(0, 16)][0] == 1

    @pl.when(deg_mode)
    def _ones():
        pltpu.sync_copy(ones_hbm, rows0)

    ys = (y0, y1, y2, y3, y4, y5)
    os = (o0, o1, o2, o3, o4, o5)
    rows = (rows0, rows1, rows2)
    gsems = (gsem0, gsem1, gsem2)
    ssems = (ssem0, ssem1, ssem2)

    def pipeline(yref, srcrows, dstrows, hooks):
        n = len(srcrows)
        g = [pltpu.make_async_copy(yref.at[srcrows[jj]],
                                   rows[jj % 3], gsems[jj % 3])
             for jj in range(n)]
        s = [pltpu.make_async_copy(rows[jj % 3],
                                   acc.at[dstrows[jj]], ssems[jj % 3])
             for jj in range(n)]
        g[0].start()
        if n > 1:
            g[1].start()
        for jj in range(n):
            if jj + 2 < n:
                if jj >= 1:
                    s[jj - 1].wait()
                pre = hooks.get(jj + 2)
                if pre is not None:
                    pre()
                g[jj + 2].start()
            g[jj].wait()
            s[jj].start(add=True)
            post = hooks.get(-(jj + 1))
            if post is not None:
                post()
        for t in range(max(0, n - 3), n):
            s[t].wait()

    @pl.when(sub < 15)
    def _zero0():
        pltpu.sync_copy(
            zero_hbm.at[pl.ds(pl.multiple_of(sub * ZROWS, 8), ZROWS)],
            acc.at[pl.ds(pl.multiple_of(sub * ZROWS, 8), ZROWS)])

    @pl.when(sub == 15)
    def _zero0_last():
        pltpu.sync_copy(zero_hbm.at[pl.ds(15 * ZROWS, ZLAST)],
                        acc.at[pl.ds(15 * ZROWS, ZLAST)])

    for si in range(NSLICE):
        active = (si == 0) | (~deg_mode)
        plsc.subcore_barrier()

        @pl.when(active & deg_mode)
        def _deg_scatter():
            def chunk(cb, carry):
                pltpu.sync_copy(dst_hbm.at[sub, cb], dstA)
                descs = [
                    pltpu.make_async_copy(rows0, acc.at[dstA.at[jj]], ssem0)
                    for jj in range(CHUNK)
                ]
                for d in descs:
                    d.start(add=True)
                for d in descs:
                    d.wait()
                return carry

            lax.fori_loop(0, NCHUNK, chunk, 0)

        @pl.when(active & (~deg_mode))
        def _scatter(si=si):
            yref = ys[si]

            pltpu.make_async_copy(src_hbm.at[sub, 0], srcA, isemA).start()
            pltpu.make_async_copy(dst_hbm.at[sub, 0], dstA, isemA).start()

            def pair(i, carry):
                cb0 = lax.mul(i, 2)
                pltpu.make_async_copy(src_hbm.at[sub, 0], srcA, isemA).wait()
                pltpu.make_async_copy(dst_hbm.at[sub, 0], dstA, isemA).wait()
                ib0 = pltpu.make_async_copy(src_hbm.at[sub, cb0 + 1], srcB, isemB)
                ib1 = pltpu.make_async_copy(dst_hbm.at[sub, cb0 + 1], dstB, isemB)
                ib0.start()
                ib1.start()

                def wait_b():
                    ib0.wait()
                    ib1.wait()

                def prefetch_next_a():
                    pltpu.make_async_copy(src_hbm.at[sub, cb0 + 2], srcA, isemA).start()
                    pltpu.make_async_copy(dst_hbm.at[sub, cb0 + 2], dstA, isemA).start()

                srcrows = ([srcA.at[jj] for jj in range(CHUNK)]
                           + [srcB.at[jj] for jj in range(CHUNK)])
                dstrows = ([dstA.at[jj] for jj in range(CHUNK)]
                           + [dstB.at[jj] for jj in range(CHUNK)])
                pipeline(yref, srcrows, dstrows,
                         {CHUNK: wait_b, -(CHUNK + 2): prefetch_next_a})
                return carry

            lax.fori_loop(0, NPAIR, pair, 0)

            pltpu.make_async_copy(src_hbm.at[sub, 0], srcA, isemA).wait()
            pltpu.make_async_copy(dst_hbm.at[sub, 0], dstA, isemA).wait()
            pipeline(yref, [srcA.at[jj] for jj in range(CHUNK)],
                     [dstA.at[jj] for jj in range(CHUNK)], {})

        plsc.subcore_barrier()

        @pl.when(active & (sub < 15))
        def _out(si=si):
            pltpu.sync_copy(
                acc.at[pl.ds(pl.multiple_of(sub * ZROWS, 8), ZROWS)],
                os[si].at[pl.ds(pl.multiple_of(sub * ZROWS, 8), ZROWS)])
            pltpu.sync_copy(
                zero_hbm.at[pl.ds(pl.multiple_of(sub * ZROWS, 8), ZROWS)],
                acc.at[pl.ds(pl.multiple_of(sub * ZROWS, 8), ZROWS)])

        @pl.when(active & (sub == 15))
        def _out_last(si=si):
            pltpu.sync_copy(acc.at[pl.ds(15 * ZROWS, ZLAST)],
                            os[si].at[pl.ds(15 * ZROWS, ZLAST)])
            pltpu.sync_copy(zero_hbm.at[pl.ds(15 * ZROWS, ZLAST)],
                            acc.at[pl.ds(15 * ZROWS, ZLAST)])

    plsc.subcore_barrier()


_agg_kernel = functools.partial(
    pl.kernel,
    out_type=[jax.ShapeDtypeStruct((N_NODES, DSL), jnp.float32)] * NSLICE,
    mesh=_sc_mesh,
    scratch_types=[
        pltpu.VMEM((16,), jnp.int32),
        pltpu.VMEM((CHUNK, EB), jnp.int32),
        pltpu.VMEM((CHUNK, EB), jnp.int32),
        pltpu.VMEM((CHUNK, EB), jnp.int32),
        pltpu.VMEM((CHUNK, EB), jnp.int32),
        pltpu.VMEM((EB, DSL), jnp.float32),
        pltpu.VMEM((EB, DSL), jnp.float32),
        pltpu.VMEM((EB, DSL), jnp.float32),
        pltpu.SemaphoreType.DMA,
        pltpu.SemaphoreType.DMA,
        pltpu.SemaphoreType.DMA,
        pltpu.SemaphoreType.DMA,
        pltpu.SemaphoreType.DMA,
        pltpu.SemaphoreType.DMA,
        pltpu.SemaphoreType.DMA,
        pltpu.SemaphoreType.DMA,
        pltpu.VMEM_SHARED((N_NODES, DSL), jnp.float32),
    ],
)(_agg_body)


def _mmu_body(x_ref, w_ref, *outs):
    h = jnp.dot(x_ref[...], w_ref[...], preferred_element_type=jnp.float32)
    for si, o in enumerate(outs):
        o[...] = h[:, si * DSL:(si + 1) * DSL]


def _mmu(x, W1):
    blk = lambda i: (i, 0)
    return pl.pallas_call(
        _mmu_body,
        grid=(MM_GRID,),
        in_specs=[
            pl.BlockSpec((MM_BLK, D), blk),
            pl.BlockSpec((D, D), lambda i: (0, 0)),
        ],
        out_specs=[pl.BlockSpec((MM_BLK, DSL), blk)] * NSLICE,
        out_shape=[jax.ShapeDtypeStruct((N_NODES, DSL), jnp.float32)] * NSLICE,
    )(x, W1)


def _scale_body(*refs):
    hs = refs[:NSLICE]
    dinv_ref = refs[NSLICE]
    outs = refs[NSLICE + 1:]
    dinv = dinv_ref[...]
    for si, o in enumerate(outs):
        o[...] = hs[si][...] * dinv


def _scale(hs, dinv):
    blk = lambda i: (i, 0)
    return pl.pallas_call(
        _scale_body,
        grid=(MM_GRID,),
        in_specs=([pl.BlockSpec((MM_BLK, DSL), blk)] * NSLICE
                  + [pl.BlockSpec((MM_BLK, 1), blk)]),
        out_specs=[pl.BlockSpec((MM_BLK, DSL), blk)] * NSLICE,
        out_shape=[jax.ShapeDtypeStruct((N_NODES, DSL), jnp.float32)] * NSLICE,
    )(*hs, dinv)


def _mm2_body(*refs):
    aggs, ys = refs[:NSLICE], refs[NSLICE:2 * NSLICE]
    w_ref, dinv_ref, b_ref = refs[2 * NSLICE:2 * NSLICE + 3]
    outs = refs[2 * NSLICE + 3:]
    agg = jnp.concatenate([a[...] for a in aggs], axis=1)
    y = jnp.concatenate([yy[...] for yy in ys], axis=1)
    dinv = dinv_ref[...]
    z = jax.nn.relu(dinv * (agg + y) + b_ref[...])
    h = jnp.dot(z, w_ref[...], preferred_element_type=jnp.float32)
    out = h * dinv
    for si, o in enumerate(outs):
        o[...] = out[:, si * DSL:(si + 1) * DSL]


def _mm2(aggs, ys, W2, dinv, b1):
    blk = lambda i: (i, 0)
    return pl.pallas_call(
        _mm2_body,
        grid=(MM_GRID,),
        in_specs=(
            [pl.BlockSpec((MM_BLK, DSL), blk)] * (2 * NSLICE)
            + [
                pl.BlockSpec((D, D), lambda i: (0, 0)),
                pl.BlockSpec((MM_BLK, 1), blk),
                pl.BlockSpec((1, D), lambda i: (0, 0)),
            ]
        ),
        out_specs=[pl.BlockSpec((MM_BLK, DSL), blk)] * NSLICE,
        out_shape=[jax.ShapeDtypeStruct((N_NODES, DSL), jnp.float32)] * NSLICE,
    )(*aggs, *ys, W2, dinv, b1)


def _fin_body(*refs):
    aggs, ys = refs[:NSLICE], refs[NSLICE:2 * NSLICE]
    dinv_ref, b_ref = refs[2 * NSLICE:2 * NSLICE + 2]
    outs = refs[2 * NSLICE + 2:]
    agg = jnp.concatenate([a[...] for a in aggs], axis=1)
    y = jnp.concatenate([yy[...] for yy in ys], axis=1)
    out = dinv_ref[...] * (agg + y) + b_ref[...]
    for si, o in enumerate(outs):
        o[...] = out[:, si * DSL:(si + 1) * DSL]


def _fin(aggs, ys, dinv, b2):
    blk = lambda i: (i, 0)
    return pl.pallas_call(
        _fin_body,
        grid=(MM_GRID,),
        in_specs=(
            [pl.BlockSpec((MM_BLK, DSL), blk)] * (2 * NSLICE)
            + [
                pl.BlockSpec((MM_BLK, 1), blk),
                pl.BlockSpec((1, D), lambda i: (0, 0)),
            ]
        ),
        out_specs=[pl.BlockSpec((MM_BLK, DSL), blk)] * NSLICE,
        out_shape=[jax.ShapeDtypeStruct((N_NODES, DSL), jnp.float32)] * NSLICE,
    )(*aggs, *ys, dinv, b2)


def _asm_body(*refs):
    ins, out_ref = refs[:NSLICE], refs[NSLICE]
    out_ref[...] = jnp.concatenate([a[...] for a in ins], axis=1)


def _assemble(slices):
    blk = lambda i: (i, 0)
    return pl.pallas_call(
        _asm_body,
        grid=(MM_GRID,),
        in_specs=[pl.BlockSpec((MM_BLK, DSL), blk)] * NSLICE,
        out_specs=pl.BlockSpec((MM_BLK, D), blk),
        out_shape=jax.ShapeDtypeStruct((N_NODES, D), jnp.float32),
    )(*slices)


@jax.jit
def kernel(x, edge_index, W1, b1, W2, b2):
    src3 = edge_index[0].reshape(NTILES, NCHUNK, CHUNK, EB)
    dst3 = edge_index[1].reshape(NTILES, NCHUNK, CHUNK, EB)

    zero_s = jnp.zeros((N_NODES, DSL), jnp.float32)
    ones_r = jnp.ones((EB, DSL), jnp.float32)
    b1r = b1.reshape(1, D)
    b2r = b2.reshape(1, D)

    h1s = _mmu(x, W1)

    def body(i, carry):
        dinv, ys = carry[0], carry[1:]
        mode = jnp.full((16,), jnp.where(i == 0, 1, 0), jnp.int32)
        aggs = _agg_kernel(mode, *ys, src3, dst3, zero_s, ones_r)

        def f_deg(aggs, ys, dinv):
            deg = aggs[0][:, :1]
            new_dinv = lax.rsqrt(deg + 1.0)
            return (new_dinv,) + tuple(_scale(h1s, new_dinv))

        def f_mid(aggs, ys, dinv):
            return (dinv,) + tuple(_mm2(aggs, ys, W2, dinv, b1r))

        def f_fin(aggs, ys, dinv):
            return (dinv,) + tuple(_fin(aggs, ys, dinv, b2r))

        return lax.switch(i, [f_deg, f_mid, f_fin], aggs, ys, dinv)

    carry0 = (jnp.ones((N_NODES, 1), jnp.float32),) + tuple(
        jnp.zeros((N_NODES, DSL), jnp.float32) for _ in range(NSLICE))
    out = lax.fori_loop(0, 3, body, carry0)
    return _assemble(out[1:])


# speedup vs baseline: 1.0103x; 1.0103x over previous
import functools

import jax
import jax.numpy as jnp
from jax import lax
from jax.experimental import pallas as pl
from jax.experimental.pallas import tpu as pltpu
from jax.experimental.pallas import tpu_sc as plsc

N_NODES = 10000
N_EDGES = 320000
D = 768

NSLICE = 6
DSL = D // NSLICE
EB = 100
NTILES = 16
ROWS_PER_TILE = N_EDGES // EB // NTILES
CHUNK = 8
NCHUNK = ROWS_PER_TILE // CHUNK
NPAIR = (NCHUNK - 1) // 2
ZROWS = 624
ZLAST = N_NODES - 15 * ZROWS
MM_BLK = 400
MM_GRID = N_NODES // MM_BLK
HROWS = 80

_sc_mesh = plsc.VectorSubcoreMesh(core_axis_name="c", subcore_axis_name="s", num_cores=1)


def _agg_body(mode_hbm, y0, y1, y2, y3, y4, y5, src_hbm, dst_hbm, zero_hbm,
              ones_hbm, o0, o1, o2, o3, o4, o5,
              mode_s, srcA, dstA, srcB, dstB, rows0, rows1, rows2,
              gsem0, gsem1, gsem2, ssem0, ssem1, ssem2, isemA, isemB, acc):
    core = lax.axis_index("c")
    sub = lax.axis_index("s")

    pltpu.sync_copy(mode_hbm, mode_s)
    deg_mode = mode_s[pl.ds(0, 16)][0] == 1

    @pl.when(deg_mode)
    def _ones():
        pltpu.sync_copy(ones_hbm, rows0)

    ys = (y0, y1, y2, y3, y4, y5)
    os = (o0, o1, o2, o3, o4, o5)
    rows = (rows0, rows1, rows2)
    gsems = (gsem0, gsem1, gsem2)
    ssems = (ssem0, ssem1, ssem2)

    def pipeline(yref, srcrows, dstrows, hooks):
        n = len(srcrows)
        g = [pltpu.make_async_copy(yref.at[srcrows[jj]],
                                   rows[jj % 3], gsems[jj % 3])
             for jj in range(n)]
        s = [pltpu.make_async_copy(rows[jj % 3],
                                   acc.at[dstrows[jj]], ssems[jj % 3])
             for jj in range(n)]
        g[0].start()
        if n > 1:
            g[1].start()
        for jj in range(n):
            if jj + 2 < n:
                if jj >= 1:
                    s[jj - 1].wait()
                pre = hooks.get(jj + 2)
                if pre is not None:
                    pre()
                g[jj + 2].start()
            g[jj].wait()
            s[jj].start(add=True)
            post = hooks.get(-(jj + 1))
            if post is not None:
                post()
        for t in range(max(0, n - 3), n):
            s[t].wait()

    @pl.when(sub < 15)
    def _zero0():
        pltpu.sync_copy(
            zero_hbm.at[pl.ds(pl.multiple_of(sub * ZROWS, 8), ZROWS)],
            acc.at[pl.ds(pl.multiple_of(sub * ZROWS, 8), ZROWS)])

    @pl.when(sub == 15)
    def _zero0_last():
        pltpu.sync_copy(zero_hbm.at[pl.ds(15 * ZROWS, ZLAST)],
                        acc.at[pl.ds(15 * ZROWS, ZLAST)])

    for si in range(NSLICE):
        active = (si == 0) | (~deg_mode)
        plsc.subcore_barrier()

        @pl.when(active & deg_mode)
        def _deg_scatter():
            def chunk(cb, carry):
                pltpu.sync_copy(dst_hbm.at[sub, cb], dstA)
                descs = [
                    pltpu.make_async_copy(rows0, acc.at[dstA.at[jj]], ssem0)
                    for jj in range(CHUNK)
                ]
                for d in descs:
                    d.start(add=True)
                for d in descs:
                    d.wait()
                return carry

            lax.fori_loop(0, NCHUNK, chunk, 0)

        @pl.when(active & (~deg_mode))
        def _scatter(si=si):
            yref = ys[si]

            pltpu.make_async_copy(src_hbm.at[sub, 0], srcA, isemA).start()
            pltpu.make_async_copy(dst_hbm.at[sub, 0], dstA, isemA).start()

            def pair(i, carry):
                cb0 = lax.mul(i, 2)
                pltpu.make_async_copy(src_hbm.at[sub, 0], srcA, isemA).wait()
                pltpu.make_async_copy(dst_hbm.at[sub, 0], dstA, isemA).wait()
                ib0 = pltpu.make_async_copy(src_hbm.at[sub, cb0 + 1], srcB, isemB)
                ib1 = pltpu.make_async_copy(dst_hbm.at[sub, cb0 + 1], dstB, isemB)
                ib0.start()
                ib1.start()

                def wait_b():
                    ib0.wait()
                    ib1.wait()

                def prefetch_next_a():
                    pltpu.make_async_copy(src_hbm.at[sub, cb0 + 2], srcA, isemA).start()
                    pltpu.make_async_copy(dst_hbm.at[sub, cb0 + 2], dstA, isemA).start()

                srcrows = ([srcA.at[jj] for jj in range(CHUNK)]
                           + [srcB.at[jj] for jj in range(CHUNK)])
                dstrows = ([dstA.at[jj] for jj in range(CHUNK)]
                           + [dstB.at[jj] for jj in range(CHUNK)])
                pipeline(yref, srcrows, dstrows,
                         {CHUNK: wait_b, -(CHUNK + 2): prefetch_next_a})
                return carry

            lax.fori_loop(0, NPAIR, pair, 0)

            pltpu.make_async_copy(src_hbm.at[sub, 0], srcA, isemA).wait()
            pltpu.make_async_copy(dst_hbm.at[sub, 0], dstA, isemA).wait()
            pipeline(yref, [srcA.at[jj] for jj in range(CHUNK)],
                     [dstA.at[jj] for jj in range(CHUNK)], {})

        plsc.subcore_barrier()

        @pl.when(active & (sub < 15))
        def _out(si=si):
            pltpu.sync_copy(
                acc.at[pl.ds(pl.multiple_of(sub * ZROWS, 8), ZROWS)],
                os[si].at[pl.ds(pl.multiple_of(sub * ZROWS, 8), ZROWS)])
            pltpu.sync_copy(
                zero_hbm.at[pl.ds(pl.multiple_of(sub * ZROWS, 8), ZROWS)],
                acc.at[pl.ds(pl.multiple_of(sub * ZROWS, 8), ZROWS)])

        @pl.when(active & (sub == 15))
        def _out_last(si=si):
            pltpu.sync_copy(acc.at[pl.ds(15 * ZROWS, ZLAST)],
                            os[si].at[pl.ds(15 * ZROWS, ZLAST)])
            pltpu.sync_copy(zero_hbm.at[pl.ds(15 * ZROWS, ZLAST)],
                            acc.at[pl.ds(15 * ZROWS, ZLAST)])

    plsc.subcore_barrier()


_agg_kernel = functools.partial(
    pl.kernel,
    out_type=[jax.ShapeDtypeStruct((N_NODES, DSL), jnp.float32)] * NSLICE,
    mesh=_sc_mesh,
    scratch_types=[
        pltpu.VMEM((16,), jnp.int32),
        pltpu.VMEM((CHUNK, EB), jnp.int32),
        pltpu.VMEM((CHUNK, EB), jnp.int32),
        pltpu.VMEM((CHUNK, EB), jnp.int32),
        pltpu.VMEM((CHUNK, EB), jnp.int32),
        pltpu.VMEM((EB, DSL), jnp.float32),
        pltpu.VMEM((EB, DSL), jnp.float32),
        pltpu.VMEM((EB, DSL), jnp.float32),
        pltpu.SemaphoreType.DMA,
        pltpu.SemaphoreType.DMA,
        pltpu.SemaphoreType.DMA,
        pltpu.SemaphoreType.DMA,
        pltpu.SemaphoreType.DMA,
        pltpu.SemaphoreType.DMA,
        pltpu.SemaphoreType.DMA,
        pltpu.SemaphoreType.DMA,
        pltpu.VMEM_SHARED((N_NODES, DSL), jnp.float32),
    ],
)(_agg_body)


def _mm1_body(x_ref, w_ref, dinv_ref, *outs):
    h = jnp.dot(x_ref[...], w_ref[...], preferred_element_type=jnp.float32)
    y = h * dinv_ref[...]
    for si, o in enumerate(outs):
        o[...] = y[:, si * DSL:(si + 1) * DSL]


def _mm1(x, W1, dinv):
    blk = lambda i: (i, 0)
    return pl.pallas_call(
        _mm1_body,
        grid=(MM_GRID,),
        in_specs=[
            pl.BlockSpec((MM_BLK, D), blk),
            pl.BlockSpec((D, D), lambda i: (0, 0)),
            pl.BlockSpec((MM_BLK, 1), blk),
        ],
        out_specs=[pl.BlockSpec((MM_BLK, DSL), blk)] * NSLICE,
        out_shape=[jax.ShapeDtypeStruct((N_NODES, DSL), jnp.float32)] * NSLICE,
    )(x, W1, dinv)


def _mm2_body(*refs):
    aggs, ys = refs[:NSLICE], refs[NSLICE:2 * NSLICE]
    w_ref, dinv_ref, b_ref = refs[2 * NSLICE:2 * NSLICE + 3]
    outs = refs[2 * NSLICE + 3:]
    agg = jnp.concatenate([a[...] for a in aggs], axis=1)
    y = jnp.concatenate([yy[...] for yy in ys], axis=1)
    dinv = dinv_ref[...]
    z = jax.nn.relu(dinv * (agg + y) + b_ref[...])
    h = jnp.dot(z, w_ref[...], preferred_element_type=jnp.float32)
    out = h * dinv
    for si, o in enumerate(outs):
        o[...] = out[:, si * DSL:(si + 1) * DSL]


def _mm2(aggs, ys, W2, dinv, b1):
    blk = lambda i: (i, 0)
    return pl.pallas_call(
        _mm2_body,
        grid=(MM_GRID,),
        in_specs=(
            [pl.BlockSpec((MM_BLK, DSL), blk)] * (2 * NSLICE)
            + [
                pl.BlockSpec((D, D), lambda i: (0, 0)),
                pl.BlockSpec((MM_BLK, 1), blk),
                pl.BlockSpec((1, D), lambda i: (0, 0)),
            ]
        ),
        out_specs=[pl.BlockSpec((MM_BLK, DSL), blk)] * NSLICE,
        out_shape=[jax.ShapeDtypeStruct((N_NODES, DSL), jnp.float32)] * NSLICE,
    )(*aggs, *ys, W2, dinv, b1)


def _fin_body(*refs):
    aggs, ys = refs[:NSLICE], refs[NSLICE:2 * NSLICE]
    dinv_ref, b_ref = refs[2 * NSLICE:2 * NSLICE + 2]
    outs = refs[2 * NSLICE + 2:]
    agg = jnp.concatenate([a[...] for a in aggs], axis=1)
    y = jnp.concatenate([yy[...] for yy in ys], axis=1)
    out = dinv_ref[...] * (agg + y) + b_ref[...]
    for si, o in enumerate(outs):
        o[...] = out[:, si * DSL:(si + 1) * DSL]


def _fin(aggs, ys, dinv, b2):
    blk = lambda i: (i, 0)
    return pl.pallas_call(
        _fin_body,
        grid=(MM_GRID,),
        in_specs=(
            [pl.BlockSpec((MM_BLK, DSL), blk)] * (2 * NSLICE)
            + [
                pl.BlockSpec((MM_BLK, 1), blk),
                pl.BlockSpec((1, D), lambda i: (0, 0)),
            ]
        ),
        out_specs=[pl.BlockSpec((MM_BLK, DSL), blk)] * NSLICE,
        out_shape=[jax.ShapeDtypeStruct((N_NODES, DSL), jnp.float32)] * NSLICE,
    )(*aggs, *ys, dinv, b2)


def _asm_body(*refs):
    ins, out_ref = refs[:NSLICE], refs[NSLICE]
    out_ref[...] = jnp.concatenate([a[...] for a in ins], axis=1)


def _assemble(slices):
    blk = lambda i: (i, 0)
    return pl.pallas_call(
        _asm_body,
        grid=(MM_GRID,),
        in_specs=[pl.BlockSpec((MM_BLK, DSL), blk)] * NSLICE,
        out_specs=pl.BlockSpec((MM_BLK, D), blk),
        out_shape=jax.ShapeDtypeStruct((N_NODES, D), jnp.float32),
    )(*slices)


@jax.jit
def kernel(x, edge_index, W1, b1, W2, b2):
    src3 = edge_index[0].reshape(NTILES, NCHUNK, CHUNK, EB)
    dst3 = edge_index[1].reshape(NTILES, NCHUNK, CHUNK, EB)

    zero_s = jnp.zeros((N_NODES, DSL), jnp.float32)
    ones_r = jnp.ones((EB, DSL), jnp.float32)
    b1r = b1.reshape(1, D)
    b2r = b2.reshape(1, D)

    def body(i, carry):
        dinv, ys = carry[0], carry[1:]
        mode = jnp.full((16,), jnp.where(i == 0, 1, 0), jnp.int32)
        aggs = _agg_kernel(mode, *ys, src3, dst3, zero_s, ones_r)

        def f_deg(aggs, ys, dinv):
            deg = aggs[0][:, :1]
            new_dinv = lax.rsqrt(deg + 1.0)
            return (new_dinv,) + tuple(_mm1(x, W1, new_dinv))

        def f_mid(aggs, ys, dinv):
            return (dinv,) + tuple(_mm2(aggs, ys, W2, dinv, b1r))

        def f_fin(aggs, ys, dinv):
            return (dinv,) + tuple(_fin(aggs, ys, dinv, b2r))

        return lax.switch(i, [f_deg, f_mid, f_fin], aggs, ys, dinv)

    carry0 = (jnp.ones((N_NODES, 1), jnp.float32),) + tuple(
        jnp.zeros((N_NODES, DSL), jnp.float32) for _ in range(NSLICE))
    out = lax.fori_loop(0, 3, body, carry0)
    return _assemble(out[1:])


# speedup vs baseline: 1.0150x; 1.0047x over previous
import functools

import jax
import jax.numpy as jnp
from jax import lax
from jax.experimental import pallas as pl
from jax.experimental.pallas import tpu as pltpu
from jax.experimental.pallas import tpu_sc as plsc

N_NODES = 10000
N_EDGES = 320000
D = 768

NSLICE = 6
DSL = D // NSLICE
EB = 80
NTILES = 16
ROWS_PER_TILE = N_EDGES // EB // NTILES
CHUNK = 10
NCHUNK = ROWS_PER_TILE // CHUNK
NPAIR = (NCHUNK - 1) // 2
ZROWS = 624
ZLAST = N_NODES - 15 * ZROWS
MM_BLK = 400
MM_GRID = N_NODES // MM_BLK
HROWS = 80

_sc_mesh = plsc.VectorSubcoreMesh(core_axis_name="c", subcore_axis_name="s", num_cores=1)


def _agg_body(mode_hbm, y0, y1, y2, y3, y4, y5, src_hbm, dst_hbm, zero_hbm,
              ones_hbm, o0, o1, o2, o3, o4, o5,
              mode_s, srcA, dstA, srcB, dstB, rows0, rows1, rows2, rows3,
              gsem0, gsem1, gsem2, gsem3, ssem0, ssem1, ssem2, ssem3,
              isemA, isemB, acc):
    core = lax.axis_index("c")
    sub = lax.axis_index("s")

    pltpu.sync_copy(mode_hbm, mode_s)
    deg_mode = mode_s[pl.ds(0, 16)][0] == 1

    @pl.when(deg_mode)
    def _ones():
        pltpu.sync_copy(ones_hbm, rows0)

    ys = (y0, y1, y2, y3, y4, y5)
    os = (o0, o1, o2, o3, o4, o5)
    rows = (rows0, rows1, rows2, rows3)
    gsems = (gsem0, gsem1, gsem2, gsem3)
    ssems = (ssem0, ssem1, ssem2, ssem3)

    def pipeline(yref, srcrows, dstrows, hooks):
        n = len(srcrows)
        g = [pltpu.make_async_copy(yref.at[srcrows[jj]],
                                   rows[jj % 4], gsems[jj % 4])
             for jj in range(n)]
        s = [pltpu.make_async_copy(rows[jj % 4],
                                   acc.at[dstrows[jj]], ssems[jj % 4])
             for jj in range(n)]
        for w in range(min(3, n)):
            g[w].start()
        for jj in range(n):
            if jj + 3 < n:
                if jj >= 1:
                    s[jj - 1].wait()
                pre = hooks.get(jj + 3)
                if pre is not None:
                    pre()
                g[jj + 3].start()
            g[jj].wait()
            s[jj].start(add=True)
            post = hooks.get(-(jj + 1))
            if post is not None:
                post()
        for t in range(max(0, n - 4), n):
            s[t].wait()

    @pl.when(sub < 15)
    def _zero0():
        pltpu.sync_copy(
            zero_hbm.at[pl.ds(pl.multiple_of(sub * ZROWS, 8), ZROWS)],
            acc.at[pl.ds(pl.multiple_of(sub * ZROWS, 8), ZROWS)])

    @pl.when(sub == 15)
    def _zero0_last():
        pltpu.sync_copy(zero_hbm.at[pl.ds(15 * ZROWS, ZLAST)],
                        acc.at[pl.ds(15 * ZROWS, ZLAST)])

    for si in range(NSLICE):
        active = (si == 0) | (~deg_mode)
        plsc.subcore_barrier()

        @pl.when(active & deg_mode)
        def _deg_scatter():
            def chunk(cb, carry):
                pltpu.sync_copy(dst_hbm.at[sub, cb], dstA)
                descs = [
                    pltpu.make_async_copy(rows0, acc.at[dstA.at[jj]], ssem0)
                    for jj in range(CHUNK)
                ]
                for d in descs:
                    d.start(add=True)
                for d in descs:
                    d.wait()
                return carry

            lax.fori_loop(0, NCHUNK, chunk, 0)

        @pl.when(active & (~deg_mode))
        def _scatter(si=si):
            yref = ys[si]

            pltpu.make_async_copy(src_hbm.at[sub, 0], srcA, isemA).start()
            pltpu.make_async_copy(dst_hbm.at[sub, 0], dstA, isemA).start()

            def pair(i, carry):
                cb0 = lax.mul(i, 2)
                pltpu.make_async_copy(src_hbm.at[sub, 0], srcA, isemA).wait()
                pltpu.make_async_copy(dst_hbm.at[sub, 0], dstA, isemA).wait()
                ib0 = pltpu.make_async_copy(src_hbm.at[sub, cb0 + 1], srcB, isemB)
                ib1 = pltpu.make_async_copy(dst_hbm.at[sub, cb0 + 1], dstB, isemB)
                ib0.start()
                ib1.start()

                def wait_b():
                    ib0.wait()
                    ib1.wait()

                def prefetch_next_a():
                    pltpu.make_async_copy(src_hbm.at[sub, cb0 + 2], srcA, isemA).start()
                    pltpu.make_async_copy(dst_hbm.at[sub, cb0 + 2], dstA, isemA).start()

                srcrows = ([srcA.at[jj] for jj in range(CHUNK)]
                           + [srcB.at[jj] for jj in range(CHUNK)])
                dstrows = ([dstA.at[jj] for jj in range(CHUNK)]
                           + [dstB.at[jj] for jj in range(CHUNK)])
                pipeline(yref, srcrows, dstrows,
                         {CHUNK: wait_b, -(CHUNK + 3): prefetch_next_a})
                return carry

            lax.fori_loop(0, NPAIR, pair, 0)

            pltpu.make_async_copy(src_hbm.at[sub, 0], srcA, isemA).wait()
            pltpu.make_async_copy(dst_hbm.at[sub, 0], dstA, isemA).wait()
            pipeline(yref, [srcA.at[jj] for jj in range(CHUNK)],
                     [dstA.at[jj] for jj in range(CHUNK)], {})

        plsc.subcore_barrier()

        @pl.when(active & (sub < 15))
        def _out(si=si):
            pltpu.sync_copy(
                acc.at[pl.ds(pl.multiple_of(sub * ZROWS, 8), ZROWS)],
                os[si].at[pl.ds(pl.multiple_of(sub * ZROWS, 8), ZROWS)])
            pltpu.sync_copy(
                zero_hbm.at[pl.ds(pl.multiple_of(sub * ZROWS, 8), ZROWS)],
                acc.at[pl.ds(pl.multiple_of(sub * ZROWS, 8), ZROWS)])

        @pl.when(active & (sub == 15))
        def _out_last(si=si):
            pltpu.sync_copy(acc.at[pl.ds(15 * ZROWS, ZLAST)],
                            os[si].at[pl.ds(15 * ZROWS, ZLAST)])
            pltpu.sync_copy(zero_hbm.at[pl.ds(15 * ZROWS, ZLAST)],
                            acc.at[pl.ds(15 * ZROWS, ZLAST)])

    plsc.subcore_barrier()


_agg_kernel = functools.partial(
    pl.kernel,
    out_type=[jax.ShapeDtypeStruct((N_NODES, DSL), jnp.float32)] * NSLICE,
    mesh=_sc_mesh,
    scratch_types=[
        pltpu.VMEM((16,), jnp.int32),
        pltpu.VMEM((CHUNK, EB), jnp.int32),
        pltpu.VMEM((CHUNK, EB), jnp.int32),
        pltpu.VMEM((CHUNK, EB), jnp.int32),
        pltpu.VMEM((CHUNK, EB), jnp.int32),
        pltpu.VMEM((EB, DSL), jnp.float32),
        pltpu.VMEM((EB, DSL), jnp.float32),
        pltpu.VMEM((EB, DSL), jnp.float32),
        pltpu.VMEM((EB, DSL), jnp.float32),
        pltpu.SemaphoreType.DMA,
        pltpu.SemaphoreType.DMA,
        pltpu.SemaphoreType.DMA,
        pltpu.SemaphoreType.DMA,
        pltpu.SemaphoreType.DMA,
        pltpu.SemaphoreType.DMA,
        pltpu.SemaphoreType.DMA,
        pltpu.SemaphoreType.DMA,
        pltpu.SemaphoreType.DMA,
        pltpu.SemaphoreType.DMA,
        pltpu.VMEM_SHARED((N_NODES, DSL), jnp.float32),
    ],
)(_agg_body)


def _mm1_body(x_ref, w_ref, dinv_ref, *outs):
    h = jnp.dot(x_ref[...], w_ref[...], preferred_element_type=jnp.float32)
    y = h * dinv_ref[...]
    for si, o in enumerate(outs):
        o[...] = y[:, si * DSL:(si + 1) * DSL]


def _mm1(x, W1, dinv):
    blk = lambda i: (i, 0)
    return pl.pallas_call(
        _mm1_body,
        grid=(MM_GRID,),
        in_specs=[
            pl.BlockSpec((MM_BLK, D), blk),
            pl.BlockSpec((D, D), lambda i: (0, 0)),
            pl.BlockSpec((MM_BLK, 1), blk),
        ],
        out_specs=[pl.BlockSpec((MM_BLK, DSL), blk)] * NSLICE,
        out_shape=[jax.ShapeDtypeStruct((N_NODES, DSL), jnp.float32)] * NSLICE,
    )(x, W1, dinv)


def _mm2_body(*refs):
    aggs, ys = refs[:NSLICE], refs[NSLICE:2 * NSLICE]
    w_ref, dinv_ref, b_ref = refs[2 * NSLICE:2 * NSLICE + 3]
    outs = refs[2 * NSLICE + 3:]
    agg = jnp.concatenate([a[...] for a in aggs], axis=1)
    y = jnp.concatenate([yy[...] for yy in ys], axis=1)
    dinv = dinv_ref[...]
    z = jax.nn.relu(dinv * (agg + y) + b_ref[...])
    h = jnp.dot(z, w_ref[...], preferred_element_type=jnp.float32)
    out = h * dinv
    for si, o in enumerate(outs):
        o[...] = out[:, si * DSL:(si + 1) * DSL]


def _mm2(aggs, ys, W2, dinv, b1):
    blk = lambda i: (i, 0)
    return pl.pallas_call(
        _mm2_body,
        grid=(MM_GRID,),
        in_specs=(
            [pl.BlockSpec((MM_BLK, DSL), blk)] * (2 * NSLICE)
            + [
                pl.BlockSpec((D, D), lambda i: (0, 0)),
                pl.BlockSpec((MM_BLK, 1), blk),
                pl.BlockSpec((1, D), lambda i: (0, 0)),
            ]
        ),
        out_specs=[pl.BlockSpec((MM_BLK, DSL), blk)] * NSLICE,
        out_shape=[jax.ShapeDtypeStruct((N_NODES, DSL), jnp.float32)] * NSLICE,
    )(*aggs, *ys, W2, dinv, b1)


def _fin_body(*refs):
    aggs, ys = refs[:NSLICE], refs[NSLICE:2 * NSLICE]
    dinv_ref, b_ref = refs[2 * NSLICE:2 * NSLICE + 2]
    outs = refs[2 * NSLICE + 2:]
    agg = jnp.concatenate([a[...] for a in aggs], axis=1)
    y = jnp.concatenate([yy[...] for yy in ys], axis=1)
    out = dinv_ref[...] * (agg + y) + b_ref[...]
    for si, o in enumerate(outs):
        o[...] = out[:, si * DSL:(si + 1) * DSL]


def _fin(aggs, ys, dinv, b2):
    blk = lambda i: (i, 0)
    return pl.pallas_call(
        _fin_body,
        grid=(MM_GRID,),
        in_specs=(
            [pl.BlockSpec((MM_BLK, DSL), blk)] * (2 * NSLICE)
            + [
                pl.BlockSpec((MM_BLK, 1), blk),
                pl.BlockSpec((1, D), lambda i: (0, 0)),
            ]
        ),
        out_specs=[pl.BlockSpec((MM_BLK, DSL), blk)] * NSLICE,
        out_shape=[jax.ShapeDtypeStruct((N_NODES, DSL), jnp.float32)] * NSLICE,
    )(*aggs, *ys, dinv, b2)


def _asm_body(*refs):
    ins, out_ref = refs[:NSLICE], refs[NSLICE]
    out_ref[...] = jnp.concatenate([a[...] for a in ins], axis=1)


def _assemble(slices):
    blk = lambda i: (i, 0)
    return pl.pallas_call(
        _asm_body,
        grid=(MM_GRID,),
        in_specs=[pl.BlockSpec((MM_BLK, DSL), blk)] * NSLICE,
        out_specs=pl.BlockSpec((MM_BLK, D), blk),
        out_shape=jax.ShapeDtypeStruct((N_NODES, D), jnp.float32),
    )(*slices)


@jax.jit
def kernel(x, edge_index, W1, b1, W2, b2):
    src3 = edge_index[0].reshape(NTILES, NCHUNK, CHUNK, EB)
    dst3 = edge_index[1].reshape(NTILES, NCHUNK, CHUNK, EB)

    zero_s = jnp.zeros((N_NODES, DSL), jnp.float32)
    ones_r = jnp.ones((EB, DSL), jnp.float32)
    b1r = b1.reshape(1, D)
    b2r = b2.reshape(1, D)

    def body(i, carry):
        dinv, ys = carry[0], carry[1:]
        mode = jnp.full((16,), jnp.where(i == 0, 1, 0), jnp.int32)
        aggs = _agg_kernel(mode, *ys, src3, dst3, zero_s, ones_r)

        def f_deg(aggs, ys, dinv):
            deg = aggs[0][:, :1]
            new_dinv = lax.rsqrt(deg + 1.0)
            return (new_dinv,) + tuple(_mm1(x, W1, new_dinv))

        def f_mid(aggs, ys, dinv):
            return (dinv,) + tuple(_mm2(aggs, ys, W2, dinv, b1r))

        def f_fin(aggs, ys, dinv):
            return (dinv,) + tuple(_fin(aggs, ys, dinv, b2r))

        return lax.switch(i, [f_deg, f_mid, f_fin], aggs, ys, dinv)

    carry0 = (jnp.ones((N_NODES, 1), jnp.float32),) + tuple(
        jnp.zeros((N_NODES, DSL), jnp.float32) for _ in range(NSLICE))
    out = lax.fori_loop(0, 3, body, carry0)
    return _assemble(out[1:])
